# baseline (speedup 1.0000x reference)
import functools

import jax
import jax.numpy as jnp
from jax import lax
from jax.experimental import pallas as pl
from jax.experimental.pallas import tpu as pltpu


TOPK = 2
BT = 256
FT = 1024


def _route_metadata(eid, E, NB, NBLK):
    N = eid.shape[0]
    eids = jnp.arange(E, dtype=jnp.int32)
    onehot = (eid[:, None] == eids[None, :]).astype(jnp.int32)
    counts = jnp.sum(onehot, axis=0)
    offsets = jnp.concatenate(
        [jnp.zeros((1,), jnp.int32), jnp.cumsum(counts).astype(jnp.int32)])
    ranks = jnp.cumsum(onehot, axis=0) - 1
    rank = jnp.take_along_axis(ranks, eid[:, None], axis=1)[:, 0]
    pos = offsets[eid] + rank
    sort_idx = jnp.zeros((N,), jnp.int32).at[pos].set(
        jnp.arange(N, dtype=jnp.int32))

    b_lo = jnp.arange(NBLK, dtype=jnp.int32)[:, None] * BT
    lo = jnp.maximum(offsets[:-1][None, :], b_lo)
    hi = jnp.minimum(offsets[1:][None, :], b_lo + BT)
    valid = (hi > lo).reshape(-1)
    dest = jnp.cumsum(valid.astype(jnp.int32)) - 1
    dest = jnp.where(valid, dest, NB)
    bfl = jnp.broadcast_to(
        jnp.arange(NBLK, dtype=jnp.int32)[:, None], (NBLK, E)).reshape(-1)
    efl = jnp.broadcast_to(eids[None, :], (NBLK, E)).reshape(-1)
    lofl = (lo - b_lo).reshape(-1)
    hifl = (hi - b_lo).reshape(-1)
    item_blk = jnp.full((NB,), NBLK - 1, jnp.int32).at[dest].set(bfl, mode='drop')
    item_e = jnp.full((NB,), E - 1, jnp.int32).at[dest].set(efl, mode='drop')
    item_lo = jnp.zeros((NB,), jnp.int32).at[dest].set(lofl, mode='drop')
    item_hi = jnp.zeros((NB,), jnp.int32).at[dest].set(hifl, mode='drop')
    return pos, sort_idx, item_blk, item_e, item_lo, item_hi


def _ffn_body(blk_s, e_s, lo_s, hi_s,
              x_ref, w_ref, W1_ref, b1_ref, W2_ref, b2_ref, out_ref,
              acc_ref, w1b_ref, w2b_ref):
    ft = pl.program_id(0)
    i = pl.program_id(1)
    nf = pl.num_programs(0)
    nb = pl.num_programs(1)
    blk = blk_s[i]
    prev = blk_s[jnp.maximum(i - 1, 0)]
    is_first = jnp.logical_or(i == 0, prev != blk)
    nxt = blk_s[jnp.minimum(i + 1, nb - 1)]
    is_last = jnp.logical_or(i == nb - 1, nxt != blk)
    lo = lo_s[i]
    hi = hi_s[i]

    recast = jnp.logical_or(i == 0, e_s[jnp.maximum(i - 1, 0)] != e_s[i])

    @pl.when(recast)
    def _():
        w1b_ref[...] = W1_ref[0].astype(jnp.bfloat16)
        w2b_ref[...] = W2_ref[0].astype(jnp.bfloat16)

    x = x_ref[...].astype(jnp.bfloat16)
    h = jnp.dot(x, w1b_ref[...], preferred_element_type=jnp.float32)
    h = jnp.maximum(h + b1_ref[0], 0.0).astype(jnp.bfloat16)
    y = jnp.dot(h, w2b_ref[...], preferred_element_type=jnp.float32)
    y = jnp.where(ft == 0, y + b2_ref[0], y)
    rows = lax.broadcasted_iota(jnp.int32, (x.shape[0], 1), 0)
    wv = w_ref[0, 0, :].reshape(x.shape[0], 1)
    wm = jnp.where(jnp.logical_and(rows >= lo, rows < hi), wv, 0.0)
    y = y * wm

    sl = pl.ds(blk * x.shape[0], x.shape[0])

    @pl.when(jnp.logical_and(ft == 0, is_first))
    def _():
        acc_ref[sl, :] = y

    @pl.when(jnp.logical_not(jnp.logical_and(ft == 0, is_first)))
    def _():
        acc_ref[sl, :] += y

    @pl.when(jnp.logical_and(ft == nf - 1, is_last))
    def _():
        out_ref[...] = acc_ref[sl, :]


def _grouped_ffn(xs, w_sorted, W1, b1, W2, b2,
                 item_blk, item_e, item_lo, item_hi, NB, NBLK):
    N, D = xs.shape
    E, _, F = W1.shape
    NF = F // FT
    w3 = w_sorted.reshape(NBLK, 1, BT)
    b1r = b1.reshape(E, 1, F)
    b2r = b2.reshape(E, 1, D)
    grid_spec = pltpu.PrefetchScalarGridSpec(
        num_scalar_prefetch=4,
        grid=(NF, NB),
        in_specs=[
            pl.BlockSpec((BT, D), lambda ft, i, blk, e, lo, hi: (blk[i], 0)),
            pl.BlockSpec((1, 1, BT), lambda ft, i, blk, e, lo, hi: (blk[i], 0, 0)),
            pl.BlockSpec((1, D, FT), lambda ft, i, blk, e, lo, hi: (e[i], 0, ft)),
            pl.BlockSpec((1, 1, FT), lambda ft, i, blk, e, lo, hi: (e[i], 0, ft)),
            pl.BlockSpec((1, FT, D), lambda ft, i, blk, e, lo, hi: (e[i], ft, 0)),
            pl.BlockSpec((1, 1, D), lambda ft, i, blk, e, lo, hi: (e[i], 0, 0)),
        ],
        out_specs=pl.BlockSpec((BT, D), lambda ft, i, blk, e, lo, hi: (blk[i], 0)),
        scratch_shapes=[pltpu.VMEM((N, D), jnp.float32),
                        pltpu.VMEM((D, FT), jnp.bfloat16),
                        pltpu.VMEM((FT, D), jnp.bfloat16)],
    )
    return pl.pallas_call(
        _ffn_body,
        grid_spec=grid_spec,
        out_shape=jax.ShapeDtypeStruct((N, D), jnp.float32),
        compiler_params=pltpu.CompilerParams(
            dimension_semantics=("arbitrary", "arbitrary")),
    )(item_blk, item_e, item_lo, item_hi, xs, w3, W1, b1r, W2, b2r)


def kernel(hidden, top_k_indices, top_k_weights, W1, b1, W2, b2):
    Bb, Ss, D = hidden.shape
    E = W1.shape[0]
    T = Bb * Ss
    K = top_k_indices.shape[-1]
    N = T * K
    NBLK = N // BT
    NB = NBLK + E - 1

    x = hidden.reshape(T, D)
    eid = top_k_indices.reshape(N).astype(jnp.int32)
    w_flat = top_k_weights.reshape(N)

    pos, sort_idx, item_blk, item_e, item_lo, item_hi = _route_metadata(
        eid, E, NB, NBLK)

    xs = x[sort_idx // K]
    w_sorted = w_flat[sort_idx]

    ys = _grouped_ffn(xs, w_sorted, W1, b1, W2, b2,
                      item_blk, item_e, item_lo, item_hi, NB, NBLK)

    out = ys[pos.reshape(T, K)].sum(axis=1)
    return out.reshape(Bb, Ss, D)


# speedup vs baseline: 1.1526x; 1.1526x over previous
import functools

import jax
import jax.numpy as jnp
from jax import lax
from jax.experimental import pallas as pl
from jax.experimental.pallas import tpu as pltpu


TOPK = 2
BT = 256
FT = 2048


def _route_metadata(eid, E, NB, NBLK):
    N = eid.shape[0]
    eids = jnp.arange(E, dtype=jnp.int32)
    onehot = (eid[:, None] == eids[None, :]).astype(jnp.int32)
    counts = jnp.sum(onehot, axis=0)
    offsets = jnp.concatenate(
        [jnp.zeros((1,), jnp.int32), jnp.cumsum(counts).astype(jnp.int32)])
    ranks = jnp.cumsum(onehot, axis=0) - 1
    rank = jnp.take_along_axis(ranks, eid[:, None], axis=1)[:, 0]
    pos = offsets[eid] + rank
    sort_idx = jnp.zeros((N,), jnp.int32).at[pos].set(
        jnp.arange(N, dtype=jnp.int32))

    b_lo = jnp.arange(NBLK, dtype=jnp.int32)[:, None] * BT
    lo = jnp.maximum(offsets[:-1][None, :], b_lo)
    hi = jnp.minimum(offsets[1:][None, :], b_lo + BT)
    valid = (hi > lo).reshape(-1)
    dest = jnp.cumsum(valid.astype(jnp.int32)) - 1
    dest = jnp.where(valid, dest, NB)
    bfl = jnp.broadcast_to(
        jnp.arange(NBLK, dtype=jnp.int32)[:, None], (NBLK, E)).reshape(-1)
    efl = jnp.broadcast_to(eids[None, :], (NBLK, E)).reshape(-1)
    lofl = (lo - b_lo).reshape(-1)
    hifl = (hi - b_lo).reshape(-1)
    item_blk = jnp.full((NB,), NBLK - 1, jnp.int32).at[dest].set(bfl, mode='drop')
    item_e = jnp.full((NB,), E - 1, jnp.int32).at[dest].set(efl, mode='drop')
    item_lo = jnp.zeros((NB,), jnp.int32).at[dest].set(lofl, mode='drop')
    item_hi = jnp.zeros((NB,), jnp.int32).at[dest].set(hifl, mode='drop')
    return pos, sort_idx, item_blk, item_e, item_lo, item_hi


def _ffn_body(blk_s, e_s, lo_s, hi_s,
              x_ref, w_ref, W1_ref, b1_ref, W2_ref, b2_ref, out_ref,
              acc_ref):
    ft = pl.program_id(0)
    i = pl.program_id(1)
    nf = pl.num_programs(0)
    nb = pl.num_programs(1)
    blk = blk_s[i]
    prev = blk_s[jnp.maximum(i - 1, 0)]
    is_first = jnp.logical_or(i == 0, prev != blk)
    nxt = blk_s[jnp.minimum(i + 1, nb - 1)]
    is_last = jnp.logical_or(i == nb - 1, nxt != blk)
    lo = lo_s[i]
    hi = hi_s[i]

    x = x_ref[...].astype(jnp.bfloat16)
    w1 = W1_ref[0].astype(jnp.bfloat16)
    h = jnp.dot(x, w1, preferred_element_type=jnp.float32)
    h = jnp.maximum(h + b1_ref[0], 0.0).astype(jnp.bfloat16)
    w2 = W2_ref[0].astype(jnp.bfloat16)
    y = jnp.dot(h, w2, preferred_element_type=jnp.float32)
    y = jnp.where(ft == 0, y + b2_ref[0], y)
    rows = lax.broadcasted_iota(jnp.int32, (x.shape[0], 1), 0)
    wv = w_ref[0, 0, :].reshape(x.shape[0], 1)
    wm = jnp.where(jnp.logical_and(rows >= lo, rows < hi), wv, 0.0)
    y = y * wm

    sl = pl.ds(blk * x.shape[0], x.shape[0])

    @pl.when(jnp.logical_and(ft == 0, is_first))
    def _():
        acc_ref[sl, :] = y

    @pl.when(jnp.logical_not(jnp.logical_and(ft == 0, is_first)))
    def _():
        acc_ref[sl, :] += y

    @pl.when(jnp.logical_and(ft == nf - 1, is_last))
    def _():
        out_ref[...] = acc_ref[sl, :]


def _grouped_ffn(xs, w_sorted, W1, b1, W2, b2,
                 item_blk, item_e, item_lo, item_hi, NB, NBLK):
    N, D = xs.shape
    E, _, F = W1.shape
    NF = F // FT
    w3 = w_sorted.reshape(NBLK, 1, BT)
    b1r = b1.reshape(E, 1, F)
    b2r = b2.reshape(E, 1, D)
    grid_spec = pltpu.PrefetchScalarGridSpec(
        num_scalar_prefetch=4,
        grid=(NF, NB),
        in_specs=[
            pl.BlockSpec((BT, D), lambda ft, i, blk, e, lo, hi: (blk[i], 0)),
            pl.BlockSpec((1, 1, BT), lambda ft, i, blk, e, lo, hi: (blk[i], 0, 0)),
            pl.BlockSpec((1, D, FT), lambda ft, i, blk, e, lo, hi: (e[i], 0, ft)),
            pl.BlockSpec((1, 1, FT), lambda ft, i, blk, e, lo, hi: (e[i], 0, ft)),
            pl.BlockSpec((1, FT, D), lambda ft, i, blk, e, lo, hi: (e[i], ft, 0)),
            pl.BlockSpec((1, 1, D), lambda ft, i, blk, e, lo, hi: (e[i], 0, 0)),
        ],
        out_specs=pl.BlockSpec((BT, D), lambda ft, i, blk, e, lo, hi: (blk[i], 0)),
        scratch_shapes=[pltpu.VMEM((N, D), jnp.float32)],
    )
    return pl.pallas_call(
        _ffn_body,
        grid_spec=grid_spec,
        out_shape=jax.ShapeDtypeStruct((N, D), jnp.float32),
        compiler_params=pltpu.CompilerParams(
            dimension_semantics=("arbitrary", "arbitrary")),
    )(item_blk, item_e, item_lo, item_hi, xs, w3, W1, b1r, W2, b2r)


def kernel(hidden, top_k_indices, top_k_weights, W1, b1, W2, b2):
    Bb, Ss, D = hidden.shape
    E = W1.shape[0]
    T = Bb * Ss
    K = top_k_indices.shape[-1]
    N = T * K
    NBLK = N // BT
    NB = NBLK + E - 1

    x = hidden.reshape(T, D)
    eid = top_k_indices.reshape(N).astype(jnp.int32)
    w_flat = top_k_weights.reshape(N)

    pos, sort_idx, item_blk, item_e, item_lo, item_hi = _route_metadata(
        eid, E, NB, NBLK)

    xs = x[sort_idx // K]
    w_sorted = w_flat[sort_idx]

    ys = _grouped_ffn(xs, w_sorted, W1, b1, W2, b2,
                      item_blk, item_e, item_lo, item_hi, NB, NBLK)

    out = ys[pos.reshape(T, K)].sum(axis=1)
    return out.reshape(Bb, Ss, D)


# speedup vs baseline: 1.3197x; 1.1450x over previous
import functools

import jax
import jax.numpy as jnp
from jax import lax
from jax.experimental import pallas as pl
from jax.experimental.pallas import tpu as pltpu
from jax.experimental.pallas import tpu_sc as plsc


TOPK = 2
BT = 256
FT = 2048

NC = 2
NS = 16
NW = NC * NS


def _route_metadata(eid, E, NB, NBLK):
    N = eid.shape[0]
    eids = jnp.arange(E, dtype=jnp.int32)
    onehot = (eid[:, None] == eids[None, :]).astype(jnp.int32)
    counts = jnp.sum(onehot, axis=0)
    offsets = jnp.concatenate(
        [jnp.zeros((1,), jnp.int32), jnp.cumsum(counts).astype(jnp.int32)])
    ranks = jnp.cumsum(onehot, axis=0) - 1
    rank = jnp.take_along_axis(ranks, eid[:, None], axis=1)[:, 0]
    pos = offsets[eid] + rank
    sort_idx = jnp.zeros((N,), jnp.int32).at[pos].set(
        jnp.arange(N, dtype=jnp.int32))

    b_lo = jnp.arange(NBLK, dtype=jnp.int32)[:, None] * BT
    lo = jnp.maximum(offsets[:-1][None, :], b_lo)
    hi = jnp.minimum(offsets[1:][None, :], b_lo + BT)
    valid = (hi > lo).reshape(-1)
    dest = jnp.cumsum(valid.astype(jnp.int32)) - 1
    dest = jnp.where(valid, dest, NB)
    bfl = jnp.broadcast_to(
        jnp.arange(NBLK, dtype=jnp.int32)[:, None], (NBLK, E)).reshape(-1)
    efl = jnp.broadcast_to(eids[None, :], (NBLK, E)).reshape(-1)
    lofl = (lo - b_lo).reshape(-1)
    hifl = (hi - b_lo).reshape(-1)
    item_blk = jnp.full((NB,), NBLK - 1, jnp.int32).at[dest].set(bfl, mode='drop')
    item_e = jnp.full((NB,), E - 1, jnp.int32).at[dest].set(efl, mode='drop')
    item_lo = jnp.zeros((NB,), jnp.int32).at[dest].set(lofl, mode='drop')
    item_hi = jnp.zeros((NB,), jnp.int32).at[dest].set(hifl, mode='drop')
    return pos, sort_idx, item_blk, item_e, item_lo, item_hi


def _ffn_body(blk_s, e_s, lo_s, hi_s,
              x_ref, w_ref, W1_ref, b1_ref, W2_ref, b2_ref, out_ref,
              acc_ref):
    ft = pl.program_id(0)
    i = pl.program_id(1)
    nf = pl.num_programs(0)
    nb = pl.num_programs(1)
    blk = blk_s[i]
    prev = blk_s[jnp.maximum(i - 1, 0)]
    is_first = jnp.logical_or(i == 0, prev != blk)
    nxt = blk_s[jnp.minimum(i + 1, nb - 1)]
    is_last = jnp.logical_or(i == nb - 1, nxt != blk)
    lo = lo_s[i]
    hi = hi_s[i]

    x = x_ref[...].astype(jnp.bfloat16)
    w1 = W1_ref[0].astype(jnp.bfloat16)
    h = jnp.dot(x, w1, preferred_element_type=jnp.float32)
    h = jnp.maximum(h + b1_ref[0], 0.0).astype(jnp.bfloat16)
    w2 = W2_ref[0].astype(jnp.bfloat16)
    y = jnp.dot(h, w2, preferred_element_type=jnp.float32)
    y = jnp.where(ft == 0, y + b2_ref[0], y)
    rows = lax.broadcasted_iota(jnp.int32, (x.shape[0], 1), 0)
    wv = w_ref[0, 0, :].reshape(x.shape[0], 1)
    wm = jnp.where(jnp.logical_and(rows >= lo, rows < hi), wv, 0.0)
    y = y * wm

    sl = pl.ds(blk * x.shape[0], x.shape[0])

    @pl.when(jnp.logical_and(ft == 0, is_first))
    def _():
        acc_ref[sl, :] = y

    @pl.when(jnp.logical_not(jnp.logical_and(ft == 0, is_first)))
    def _():
        acc_ref[sl, :] += y

    @pl.when(jnp.logical_and(ft == nf - 1, is_last))
    def _():
        out_ref[...] = acc_ref[sl, :]


def _grouped_ffn(xs, w_sorted, W1, b1, W2, b2,
                 item_blk, item_e, item_lo, item_hi, NB, NBLK):
    N, D = xs.shape
    E, _, F = W1.shape
    NF = F // FT
    w3 = w_sorted.reshape(NBLK, 1, BT)
    b1r = b1.reshape(E, 1, F)
    b2r = b2.reshape(E, 1, D)
    grid_spec = pltpu.PrefetchScalarGridSpec(
        num_scalar_prefetch=4,
        grid=(NF, NB),
        in_specs=[
            pl.BlockSpec((BT, D), lambda ft, i, blk, e, lo, hi: (blk[i], 0)),
            pl.BlockSpec((1, 1, BT), lambda ft, i, blk, e, lo, hi: (blk[i], 0, 0)),
            pl.BlockSpec((1, D, FT), lambda ft, i, blk, e, lo, hi: (e[i], 0, ft)),
            pl.BlockSpec((1, 1, FT), lambda ft, i, blk, e, lo, hi: (e[i], 0, ft)),
            pl.BlockSpec((1, FT, D), lambda ft, i, blk, e, lo, hi: (e[i], ft, 0)),
            pl.BlockSpec((1, 1, D), lambda ft, i, blk, e, lo, hi: (e[i], 0, 0)),
        ],
        out_specs=pl.BlockSpec((BT, D), lambda ft, i, blk, e, lo, hi: (blk[i], 0)),
        scratch_shapes=[pltpu.VMEM((N, D), jnp.float32)],
    )
    return pl.pallas_call(
        _ffn_body,
        grid_spec=grid_spec,
        out_shape=jax.ShapeDtypeStruct((N, D), jnp.float32),
        compiler_params=pltpu.CompilerParams(
            dimension_semantics=("arbitrary", "arbitrary")),
    )(item_blk, item_e, item_lo, item_hi, xs, w3, W1, b1r, W2, b2r)


def _sc_gather(x, tok_sorted):
    T, D = x.shape
    N = tok_sorted.shape[0]
    RW = N // NW
    C = 64
    NCH = RW // C
    mesh = plsc.VectorSubcoreMesh(core_axis_name="c", subcore_axis_name="s",
                                  num_cores=NC, num_subcores=NS)

    @functools.partial(
        pl.kernel, mesh=mesh,
        out_type=jax.ShapeDtypeStruct((N, D), jnp.float32),
        scratch_types=[
            pltpu.VMEM((NCH, C), jnp.int32),
            pltpu.VMEM((C, D), jnp.float32),
            pltpu.SemaphoreType.DMA,
        ],
    )
    def gk(x_hbm, tok_hbm, out_hbm, idx_v, rows_v, sem):
        wid = lax.axis_index("s") * NC + lax.axis_index("c")
        base = wid * RW
        for c in range(NCH):
            pltpu.sync_copy(tok_hbm.at[pl.ds(base + c * C, C)], idx_v.at[c])
        for c in range(NCH):
            pltpu.async_copy(x_hbm.at[idx_v.at[c]], rows_v, sem).wait()
            pltpu.sync_copy(rows_v, out_hbm.at[pl.ds(base + c * C, C)])

    return gk(x, tok_sorted)


def _sc_combine(ys, pos):
    N, D = ys.shape
    T = N // TOPK
    TPW = T // NW
    CT = 16
    NCH = TPW // CT
    NV = D // 16
    mesh = plsc.VectorSubcoreMesh(core_axis_name="c", subcore_axis_name="s",
                                  num_cores=NC, num_subcores=NS)

    @functools.partial(
        pl.kernel, mesh=mesh,
        out_type=jax.ShapeDtypeStruct((T, D), jnp.float32),
        scratch_types=[
            pltpu.VMEM((NCH, TOPK * CT), jnp.int32),
            pltpu.VMEM((TOPK * CT, D), jnp.float32),
            pltpu.VMEM((CT, D), jnp.float32),
            pltpu.SemaphoreType.DMA,
        ],
    )
    def ck(ys_hbm, pos_hbm, out_hbm, idx_v, rows_v, acc_v, sem):
        wid = lax.axis_index("s") * NC + lax.axis_index("c")
        base = wid * TPW
        for c in range(NCH):
            pltpu.sync_copy(
                pos_hbm.at[pl.ds((base + c * CT) * TOPK, CT * TOPK)],
                idx_v.at[c])
        for c in range(NCH):
            pltpu.async_copy(ys_hbm.at[idx_v.at[c]], rows_v, sem).wait()

            def body(v, _):
                t = v // NV
                j = v - t * NV
                sl = pl.ds(j * 16, 16)
                acc_v[t, sl] = rows_v[2 * t, sl] + rows_v[2 * t + 1, sl]
                return 0

            lax.fori_loop(0, CT * NV, body, 0, unroll=4)
            pltpu.sync_copy(acc_v,
                            out_hbm.at[pl.ds(base + c * CT, CT)])

    return ck(ys, pos)


def kernel(hidden, top_k_indices, top_k_weights, W1, b1, W2, b2):
    Bb, Ss, D = hidden.shape
    E = W1.shape[0]
    T = Bb * Ss
    K = top_k_indices.shape[-1]
    N = T * K
    NBLK = N // BT
    NB = NBLK + E - 1

    x = hidden.reshape(T, D)
    eid = top_k_indices.reshape(N).astype(jnp.int32)
    w_flat = top_k_weights.reshape(N)

    pos, sort_idx, item_blk, item_e, item_lo, item_hi = _route_metadata(
        eid, E, NB, NBLK)

    xs = _sc_gather(x, sort_idx // K)
    w_sorted = w_flat[sort_idx]

    ys = _grouped_ffn(xs, w_sorted, W1, b1, W2, b2,
                      item_blk, item_e, item_lo, item_hi, NB, NBLK)

    out = _sc_combine(ys, pos)
    return out.reshape(Bb, Ss, D)


# speedup vs baseline: 1.3364x; 1.0127x over previous
import functools

import jax
import jax.numpy as jnp
from jax import lax
from jax.experimental import pallas as pl
from jax.experimental.pallas import tpu as pltpu
from jax.experimental.pallas import tpu_sc as plsc


TOPK = 2
BT = 256
FT = 2048

NC = 2
NS = 16
NW = NC * NS


def _route_metadata(eid, E, NB, NBLK):
    N = eid.shape[0]
    eids = jnp.arange(E, dtype=jnp.int32)
    onehot = (eid[:, None] == eids[None, :]).astype(jnp.int32)
    counts = jnp.sum(onehot, axis=0)
    offsets = jnp.concatenate(
        [jnp.zeros((1,), jnp.int32), jnp.cumsum(counts).astype(jnp.int32)])
    ranks = jnp.cumsum(onehot, axis=0) - 1
    rank = jnp.take_along_axis(ranks, eid[:, None], axis=1)[:, 0]
    pos = offsets[eid] + rank
    sort_idx = jnp.zeros((N,), jnp.int32).at[pos].set(
        jnp.arange(N, dtype=jnp.int32))

    b_lo = jnp.arange(NBLK, dtype=jnp.int32)[:, None] * BT
    lo = jnp.maximum(offsets[:-1][None, :], b_lo)
    hi = jnp.minimum(offsets[1:][None, :], b_lo + BT)
    valid = (hi > lo).reshape(-1)
    dest = jnp.cumsum(valid.astype(jnp.int32)) - 1
    dest = jnp.where(valid, dest, NB)
    bfl = jnp.broadcast_to(
        jnp.arange(NBLK, dtype=jnp.int32)[:, None], (NBLK, E)).reshape(-1)
    efl = jnp.broadcast_to(eids[None, :], (NBLK, E)).reshape(-1)
    lofl = (lo - b_lo).reshape(-1)
    hifl = (hi - b_lo).reshape(-1)
    item_blk = jnp.full((NB,), NBLK - 1, jnp.int32).at[dest].set(bfl, mode='drop')
    item_e = jnp.full((NB,), E - 1, jnp.int32).at[dest].set(efl, mode='drop')
    item_lo = jnp.zeros((NB,), jnp.int32).at[dest].set(lofl, mode='drop')
    item_hi = jnp.zeros((NB,), jnp.int32).at[dest].set(hifl, mode='drop')
    return pos, sort_idx, item_blk, item_e, item_lo, item_hi


def _ffn_body(blk_s, e_s, lo_s, hi_s,
              x_ref, w_ref, W1_ref, b1_ref, W2_ref, b2_ref, out_ref,
              acc_ref):
    ft = pl.program_id(0)
    i = pl.program_id(1)
    nf = pl.num_programs(0)
    nb = pl.num_programs(1)
    blk = blk_s[i]
    prev = blk_s[jnp.maximum(i - 1, 0)]
    is_first = jnp.logical_or(i == 0, prev != blk)
    nxt = blk_s[jnp.minimum(i + 1, nb - 1)]
    is_last = jnp.logical_or(i == nb - 1, nxt != blk)
    lo = lo_s[i]
    hi = hi_s[i]

    x = x_ref[...].astype(jnp.bfloat16)
    w1 = W1_ref[0].astype(jnp.bfloat16)
    h = jnp.dot(x, w1, preferred_element_type=jnp.float32)
    h = jnp.maximum(h + b1_ref[0], 0.0).astype(jnp.bfloat16)
    w2 = W2_ref[0].astype(jnp.bfloat16)
    y = jnp.dot(h, w2, preferred_element_type=jnp.float32)
    y = jnp.where(ft == 0, y + b2_ref[0], y)
    rows = lax.broadcasted_iota(jnp.int32, (x.shape[0], 1), 0)
    wv = w_ref[0, 0, :].reshape(x.shape[0], 1)
    wm = jnp.where(jnp.logical_and(rows >= lo, rows < hi), wv, 0.0)
    y = y * wm

    sl = pl.ds(blk * x.shape[0], x.shape[0])

    @pl.when(jnp.logical_and(ft == 0, is_first))
    def _():
        acc_ref[sl, :] = y

    @pl.when(jnp.logical_not(jnp.logical_and(ft == 0, is_first)))
    def _():
        acc_ref[sl, :] += y

    @pl.when(jnp.logical_and(ft == nf - 1, is_last))
    def _():
        out_ref[...] = acc_ref[sl, :]


def _grouped_ffn(xs, w_sorted, W1, b1, W2, b2,
                 item_blk, item_e, item_lo, item_hi, NB, NBLK):
    N, D = xs.shape
    E, _, F = W1.shape
    NF = F // FT
    w3 = w_sorted.reshape(NBLK, 1, BT)
    b1r = b1.reshape(E, 1, F)
    b2r = b2.reshape(E, 1, D)
    out_map = lambda ft, i, blk, e, lo, hi: (
        jnp.where(ft == NF - 1, blk[i], 0), 0)
    grid_spec = pltpu.PrefetchScalarGridSpec(
        num_scalar_prefetch=4,
        grid=(NF, NB),
        in_specs=[
            pl.BlockSpec((BT, D), lambda ft, i, blk, e, lo, hi: (blk[i], 0)),
            pl.BlockSpec((1, 1, BT), lambda ft, i, blk, e, lo, hi: (blk[i], 0, 0)),
            pl.BlockSpec((1, D, FT), lambda ft, i, blk, e, lo, hi: (e[i], 0, ft)),
            pl.BlockSpec((1, 1, FT), lambda ft, i, blk, e, lo, hi: (e[i], 0, ft)),
            pl.BlockSpec((1, FT, D), lambda ft, i, blk, e, lo, hi: (e[i], ft, 0)),
            pl.BlockSpec((1, 1, D), lambda ft, i, blk, e, lo, hi: (e[i], 0, 0)),
        ],
        out_specs=pl.BlockSpec((BT, D), out_map),
        scratch_shapes=[pltpu.VMEM((N, D), jnp.float32)],
    )
    return pl.pallas_call(
        _ffn_body,
        grid_spec=grid_spec,
        out_shape=jax.ShapeDtypeStruct((N, D), jnp.float32),
        compiler_params=pltpu.CompilerParams(
            dimension_semantics=("arbitrary", "arbitrary")),
    )(item_blk, item_e, item_lo, item_hi, xs, w3, W1, b1r, W2, b2r)


def _sc_gather(x, tok_sorted):
    T, D = x.shape
    N = tok_sorted.shape[0]
    RW = N // NW
    C = 64
    NCH = RW // C
    mesh = plsc.VectorSubcoreMesh(core_axis_name="c", subcore_axis_name="s",
                                  num_cores=NC, num_subcores=NS)

    @functools.partial(
        pl.kernel, mesh=mesh,
        out_type=jax.ShapeDtypeStruct((N, D), jnp.float32),
        scratch_types=[
            pltpu.VMEM((NCH, C), jnp.int32),
            pltpu.VMEM((C, D), jnp.float32),
            pltpu.SemaphoreType.DMA,
        ],
    )
    def gk(x_hbm, tok_hbm, out_hbm, idx_v, rows_v, sem):
        wid = lax.axis_index("s") * NC + lax.axis_index("c")
        base = wid * RW
        for c in range(NCH):
            pltpu.sync_copy(tok_hbm.at[pl.ds(base + c * C, C)], idx_v.at[c])
        for c in range(NCH):
            pltpu.async_copy(x_hbm.at[idx_v.at[c]], rows_v, sem).wait()
            pltpu.sync_copy(rows_v, out_hbm.at[pl.ds(base + c * C, C)])

    return gk(x, tok_sorted)


def _sc_combine(ys, pos):
    N, D = ys.shape
    T = N // TOPK
    TPW = T // NW
    CT = 16
    NCH = TPW // CT
    NV = D // 16
    mesh = plsc.VectorSubcoreMesh(core_axis_name="c", subcore_axis_name="s",
                                  num_cores=NC, num_subcores=NS)

    @functools.partial(
        pl.kernel, mesh=mesh,
        out_type=jax.ShapeDtypeStruct((T, D), jnp.float32),
        scratch_types=[
            pltpu.VMEM((NCH, TOPK * CT), jnp.int32),
            pltpu.VMEM((TOPK * CT, D), jnp.float32),
            pltpu.VMEM((CT, D), jnp.float32),
            pltpu.SemaphoreType.DMA,
        ],
    )
    def ck(ys_hbm, pos_hbm, out_hbm, idx_v, rows_v, acc_v, sem):
        wid = lax.axis_index("s") * NC + lax.axis_index("c")
        base = wid * TPW
        for c in range(NCH):
            pltpu.sync_copy(
                pos_hbm.at[pl.ds((base + c * CT) * TOPK, CT * TOPK)],
                idx_v.at[c])
        for c in range(NCH):
            pltpu.async_copy(ys_hbm.at[idx_v.at[c]], rows_v, sem).wait()

            def body(v, _):
                t = v // NV
                j = v - t * NV
                sl = pl.ds(j * 16, 16)
                acc_v[t, sl] = rows_v[2 * t, sl] + rows_v[2 * t + 1, sl]
                return 0

            lax.fori_loop(0, CT * NV, body, 0, unroll=4)
            pltpu.sync_copy(acc_v,
                            out_hbm.at[pl.ds(base + c * CT, CT)])

    return ck(ys, pos)


def kernel(hidden, top_k_indices, top_k_weights, W1, b1, W2, b2):
    Bb, Ss, D = hidden.shape
    E = W1.shape[0]
    T = Bb * Ss
    K = top_k_indices.shape[-1]
    N = T * K
    NBLK = N // BT
    NB = NBLK + E - 1

    x = hidden.reshape(T, D)
    eid = top_k_indices.reshape(N).astype(jnp.int32)
    w_flat = top_k_weights.reshape(N)

    pos, sort_idx, item_blk, item_e, item_lo, item_hi = _route_metadata(
        eid, E, NB, NBLK)

    xs = _sc_gather(x, sort_idx // K)
    w_sorted = w_flat[sort_idx]

    ys = _grouped_ffn(xs, w_sorted, W1, b1, W2, b2,
                      item_blk, item_e, item_lo, item_hi, NB, NBLK)

    out = _sc_combine(ys, pos)
    return out.reshape(Bb, Ss, D)
